# baseline (speedup 1.0000x reference)
import functools

import jax
import jax.numpy as jnp
from jax import lax
from jax.experimental import pallas as pl
from jax.experimental.pallas import tpu as pltpu
from jax.experimental.pallas import tpu_sc as plsc

NC = 2
NS = 16
NW = NC * NS

B = 16384
D_OP = 24
D_SH = 8
D = D_OP + D_SH
BPW = B // NW
L = 16
NCHUNK = 4
CH = BPW // NCHUNK

TC_GRID = 8
TC_ROWS = B // TC_GRID


_GATHER_DNUMS = lax.GatherDimensionNumbers(
    offset_dims=(), collapsed_slice_dims=(0,), start_index_map=(0,))


def _take(v, idx):
    return lax.gather(v, idx[:, None], dimension_numbers=_GATHER_DNUMS,
                      slice_sizes=(1,),
                      mode=lax.GatherScatterMode.PROMISE_IN_BOUNDS)


def _proj_body(sv_ref, w_ref, b_ref, out_ref):
    e = jnp.dot(sv_ref[...], w_ref[...],
                preferred_element_type=jnp.float32) + b_ref[...]
    e3 = e.reshape(TC_ROWS // 2, 2, D_SH)
    out_ref[:, :D_SH] = e3[:, 0, :]
    out_ref[:, D_SH:] = e3[:, 1, :]


@jax.jit
def _tc_proj(shape_vals, lin_W, lin_b2):
    return pl.pallas_call(
        _proj_body,
        grid=(TC_GRID,),
        in_specs=[
            pl.BlockSpec((TC_ROWS, 2), lambda i: (i, 0)),
            pl.BlockSpec((2, D_SH), lambda i: (0, 0)),
            pl.BlockSpec((1, D_SH), lambda i: (0, 0)),
        ],
        out_specs=pl.BlockSpec((TC_ROWS // 2, 2 * D_SH), lambda i: (i, 0)),
        out_shape=jax.ShapeDtypeStruct((B // 2, 2 * D_SH), jnp.float32),
    )(shape_vals, lin_W, lin_b2)


def _body(node_hbm, emb_hbm, table_hbm, out_hbm,
          idx_v, emb_v, comb_v, drain_v, *sems):
    wid = lax.axis_index("s") * NC + lax.axis_index("c")
    base = wid * BPW

    pltpu.sync_copy(node_hbm.at[pl.ds(base, BPW)], idx_v)
    pltpu.sync_copy(emb_hbm.at[pl.ds(wid * (BPW // 2), BPW // 2), :], emb_v)

    def fire_it(sem):
        def go(i, c):
            v = idx_v[pl.ds(L * i, L)]
            for k in range(L):
                pltpu.async_copy(table_hbm.at[v[k]],
                                 comb_v.at[L * i + k, pl.ds(0, D_OP)], sem)
            return c
        return go

    for j in range(NCHUNK):
        lax.fori_loop(j * (CH // L), (j + 1) * (CH // L),
                      fire_it(sems[j]), 0)

    iota = lax.iota(jnp.int32, 16)
    lo = iota & 7
    hi_sel = iota < 8
    shuf_hi = lo + 8

    def group_it(g, c):
        for k in range(L):
            r = L * g + k
            mrow = emb_v[8 * g + k // 2, :]
            vh = comb_v[r, pl.ds(8, 16)]
            vt = _take(vh, shuf_hi)
            e = _take(mrow, lo + 8 * (k % 2))
            comb_v[r, pl.ds(16, 16)] = jnp.where(hi_sel, vt, e)
        return c

    for j in range(NCHUNK):
        pltpu.make_async_copy(
            node_hbm.at[pl.ds(0, CH * D_OP)], drain_v, sems[j]).wait()
        lax.fori_loop(j * (CH // L), (j + 1) * (CH // L), group_it, 0)

    pltpu.sync_copy(comb_v, out_hbm.at[pl.ds(base, BPW), :])


@functools.lru_cache(maxsize=1)
def _sc_call():
    return pl.kernel(
        _body,
        out_type=jax.ShapeDtypeStruct((B, D), jnp.float32),
        mesh=plsc.VectorSubcoreMesh(core_axis_name="c", subcore_axis_name="s",
                                    num_cores=NC, num_subcores=NS),
        scratch_types=[
            pltpu.VMEM((BPW,), jnp.int32),
            pltpu.VMEM((BPW // 2, 2 * D_SH), jnp.float32),
            pltpu.VMEM((BPW, D), jnp.float32),
            pltpu.VMEM((CH * D_OP,), jnp.int32),
        ] + [pltpu.SemaphoreType.DMA] * NCHUNK,
        compiler_params=pltpu.CompilerParams(use_tc_tiling_on_sc=True),
    )


@jax.jit
def kernel(node_inds, shape_vals, op_table, lin_W, lin_b):
    emb16 = _tc_proj(shape_vals, lin_W, lin_b.reshape(1, D_SH))
    return _sc_call()(node_inds.astype(jnp.int32), emb16, op_table)
